# baseline (speedup 1.0000x reference)
import jax
import jax.numpy as jnp
from jax import lax
from jax.experimental import pallas as pl
from jax.experimental.pallas import tpu as pltpu
from jax.experimental.pallas import tpu_sc as plsc

_INPUT_SCALE = 0.0627
_QMAX = 255.0

_ROWS = 128
_COLS = 32768
_LANES = 16
_NC = 2
_NS = 16
_NW = _NC * _NS
_ROWS_PER_W = _ROWS // _NW
_NBUF = 3
_UNROLL = 16


def _body(in_hbm, lut_hbm, out_hbm,
          buf0, buf1, buf2, lut, ebuf,
          i0, i1, i2, o0, o1, o2):
    wid = lax.axis_index("s") * _NC + lax.axis_index("c")
    base = wid * _ROWS_PER_W
    pltpu.sync_copy(lut_hbm, lut)

    bufs = (buf0, buf1, buf2)
    isems = (i0, i1, i2)
    osems = (o0, o1, o2)
    in_cp = [None] * _ROWS_PER_W
    out_cp = [None] * _ROWS_PER_W

    in_cp[0] = pltpu.async_copy(in_hbm.at[base], bufs[0], isems[0])
    in_cp[1] = pltpu.async_copy(in_hbm.at[base + 1], bufs[1], isems[1])

    for rr in range(_ROWS_PER_W):
        b = rr % _NBUF
        buf = bufs[b]
        in_cp[rr].wait()

        @plsc.parallel_loop(0, _COLS, 2 * _LANES, unroll=_UNROLL,
                            carry=jnp.zeros((_LANES,), jnp.float32))
        def p1(i, acc, buf=buf):
            x0 = buf[pl.ds(i, _LANES)]
            x1 = buf[pl.ds(i + _LANES, _LANES)]
            q0 = jnp.clip(x0 / _INPUT_SCALE, 0.0, _QMAX).astype(jnp.int32)
            q1 = jnp.clip(x1 / _INPUT_SCALE, 0.0, _QMAX).astype(jnp.int32)
            e0 = plsc.load_gather(lut, [q0])
            e1 = plsc.load_gather(lut, [q1])
            packed = plsc.pack(e0, e1, format=plsc.PackFormat.INTERLEAVED)
            ebuf[pl.ds(i // 2, _LANES)] = plsc.bitcast(packed, jnp.float32)
            return acc + e0 + e1

        total = jnp.broadcast_to(jnp.sum(p1), (_LANES,))
        inv = jnp.ones((_LANES,), jnp.float32) / total

        @plsc.parallel_loop(0, _COLS, 2 * _LANES, unroll=_UNROLL)
        def p2(i, buf=buf, inv=inv):
            packed = plsc.bitcast(ebuf[pl.ds(i // 2, _LANES)], jnp.bfloat16)
            e0, e1 = plsc.unpack(packed, format=plsc.PackFormat.INTERLEAVED)
            buf[pl.ds(i, _LANES)] = e0 * inv
            buf[pl.ds(i + _LANES, _LANES)] = e1 * inv

        if rr >= 1:
            out_cp[rr - 1].wait()
        if rr + 2 < _ROWS_PER_W:
            nb = (rr + 2) % _NBUF
            in_cp[rr + 2] = pltpu.async_copy(
                in_hbm.at[base + rr + 2], bufs[nb], isems[nb])
        out_cp[rr] = pltpu.async_copy(buf, out_hbm.at[base + rr], osems[b])

    out_cp[_ROWS_PER_W - 1].wait()


def kernel(input, array):
    mesh = plsc.VectorSubcoreMesh(core_axis_name="c", subcore_axis_name="s")
    f = pl.kernel(
        _body,
        mesh=mesh,
        out_type=jax.ShapeDtypeStruct((_ROWS, _COLS), jnp.float32),
        scratch_types=[
            pltpu.VMEM((_COLS,), jnp.float32),
            pltpu.VMEM((_COLS,), jnp.float32),
            pltpu.VMEM((_COLS,), jnp.float32),
            pltpu.VMEM((256,), jnp.float32),
            pltpu.VMEM((_COLS // 2,), jnp.float32),
            pltpu.SemaphoreType.DMA,
            pltpu.SemaphoreType.DMA,
            pltpu.SemaphoreType.DMA,
            pltpu.SemaphoreType.DMA,
            pltpu.SemaphoreType.DMA,
            pltpu.SemaphoreType.DMA,
        ],
        compiler_params=pltpu.CompilerParams(needs_layout_passes=False),
    )
    return f(input, array)


# speedup vs baseline: 1.0945x; 1.0945x over previous
import jax
import jax.numpy as jnp
from jax import lax
from jax.experimental import pallas as pl
from jax.experimental.pallas import tpu as pltpu
from jax.experimental.pallas import tpu_sc as plsc

_INPUT_SCALE = 0.0627
_QMAX = 255.0

_ROWS = 128
_COLS = 32768
_LANES = 16
_NC = 2
_NS = 16
_NW = _NC * _NS
_ROWS_PER_W = _ROWS // _NW
_NBUF = 3
_UNROLL = 8


def _body(in_hbm, lut_hbm, out_hbm,
          buf0, buf1, buf2, lut,
          i0, i1, i2, o0, o1, o2):
    wid = lax.axis_index("s") * _NC + lax.axis_index("c")
    base = wid * _ROWS_PER_W

    bufs = (buf0, buf1, buf2)
    isems = (i0, i1, i2)
    osems = (o0, o1, o2)
    in_cp = [None] * _ROWS_PER_W
    out_cp = [None] * _ROWS_PER_W

    in_cp[0] = pltpu.async_copy(in_hbm.at[base], bufs[0], isems[0])
    lut_cp = pltpu.async_copy(lut_hbm, lut, o0)
    in_cp[1] = pltpu.async_copy(in_hbm.at[base + 1], bufs[1], isems[1])
    lut_cp.wait()

    for rr in range(_ROWS_PER_W):
        b = rr % _NBUF
        buf = bufs[b]
        in_cp[rr].wait()

        @plsc.parallel_loop(0, _COLS, _LANES, unroll=_UNROLL,
                            carry=jnp.zeros((_LANES,), jnp.float32))
        def p1(i, acc, buf=buf):
            x = buf[pl.ds(i, _LANES)]
            q = jnp.clip(x / _INPUT_SCALE, 0.0, _QMAX).astype(jnp.int32)
            e = plsc.load_gather(lut, [q])
            buf[pl.ds(i, _LANES)] = e
            return acc + e

        total = jnp.broadcast_to(jnp.sum(p1), (_LANES,))
        inv = jnp.ones((_LANES,), jnp.float32) / total

        @plsc.parallel_loop(0, _COLS, _LANES, unroll=_UNROLL)
        def p2(i, buf=buf, inv=inv):
            buf[pl.ds(i, _LANES)] = buf[pl.ds(i, _LANES)] * inv

        if rr >= 1:
            out_cp[rr - 1].wait()
        if rr + 2 < _ROWS_PER_W:
            nb = (rr + 2) % _NBUF
            in_cp[rr + 2] = pltpu.async_copy(
                in_hbm.at[base + rr + 2], bufs[nb], isems[nb])
        out_cp[rr] = pltpu.async_copy(buf, out_hbm.at[base + rr], osems[b])

    out_cp[_ROWS_PER_W - 1].wait()


def kernel(input, array):
    mesh = plsc.VectorSubcoreMesh(core_axis_name="c", subcore_axis_name="s")
    f = pl.kernel(
        _body,
        mesh=mesh,
        out_type=jax.ShapeDtypeStruct((_ROWS, _COLS), jnp.float32),
        scratch_types=[
            pltpu.VMEM((_COLS,), jnp.float32),
            pltpu.VMEM((_COLS,), jnp.float32),
            pltpu.VMEM((_COLS,), jnp.float32),
            pltpu.VMEM((256,), jnp.float32),
            pltpu.SemaphoreType.DMA,
            pltpu.SemaphoreType.DMA,
            pltpu.SemaphoreType.DMA,
            pltpu.SemaphoreType.DMA,
            pltpu.SemaphoreType.DMA,
            pltpu.SemaphoreType.DMA,
        ],
        compiler_params=pltpu.CompilerParams(needs_layout_passes=False),
    )
    return f(input, array)


# speedup vs baseline: 1.1030x; 1.0078x over previous
import jax
import jax.numpy as jnp
from jax import lax
from jax.experimental import pallas as pl
from jax.experimental.pallas import tpu as pltpu
from jax.experimental.pallas import tpu_sc as plsc

_INPUT_SCALE = 0.0627
_QMAX = 255.0

_ROWS = 128
_COLS = 32768
_HALF = _COLS // 2
_LANES = 16
_NC = 2
_NS = 16
_NW = _NC * _NS
_ROWS_PER_W = _ROWS // _NW
_NBUF = 3
_UNROLL = 8


def _quantize_gather_sum(buf, lut, lo, hi, acc0):
    @plsc.parallel_loop(lo, hi, _LANES, unroll=_UNROLL, carry=acc0)
    def p1(i, acc):
        x = buf[pl.ds(i, _LANES)]
        q = jnp.clip(x / _INPUT_SCALE, 0.0, _QMAX).astype(jnp.int32)
        e = plsc.load_gather(lut, [q])
        buf[pl.ds(i, _LANES)] = e
        return acc + e

    return p1


def _scale(buf, inv, lo, hi):
    @plsc.parallel_loop(lo, hi, _LANES, unroll=_UNROLL)
    def p2(i):
        buf[pl.ds(i, _LANES)] = buf[pl.ds(i, _LANES)] * inv


def _body(in_hbm, lut_hbm, out_hbm,
          buf0, buf1, buf2, lut,
          i0, i1, i2, o0, o1, o2):
    wid = lax.axis_index("s") * _NC + lax.axis_index("c")
    base = wid * _ROWS_PER_W

    bufs = (buf0, buf1, buf2)
    isems = (i0, i1, i2)
    osems = (o0, o1, o2)
    in_cp = [None] * _ROWS_PER_W
    out_cp = [None] * _ROWS_PER_W
    zero = jnp.zeros((_LANES,), jnp.float32)

    cp_h0 = pltpu.async_copy(in_hbm.at[base, pl.ds(0, _HALF)],
                             buf0.at[pl.ds(0, _HALF)], isems[0])
    lut_cp = pltpu.async_copy(lut_hbm, lut, o0)
    cp_h1 = pltpu.async_copy(in_hbm.at[base, pl.ds(_HALF, _HALF)],
                             buf0.at[pl.ds(_HALF, _HALF)], o1)
    in_cp[1] = pltpu.async_copy(in_hbm.at[base + 1], bufs[1], isems[1])
    lut_cp.wait()

    for rr in range(_ROWS_PER_W):
        b = rr % _NBUF
        buf = bufs[b]
        if rr == 0:
            cp_h0.wait()
            acc = _quantize_gather_sum(buf, lut, 0, _HALF, zero)
            cp_h1.wait()
            acc = _quantize_gather_sum(buf, lut, _HALF, _COLS, acc)
        else:
            in_cp[rr].wait()
            acc = _quantize_gather_sum(buf, lut, 0, _COLS, zero)

        total = jnp.broadcast_to(jnp.sum(acc), (_LANES,))
        inv = jnp.ones((_LANES,), jnp.float32) / total

        if rr >= 1:
            out_cp[rr - 1].wait()
        if rr + 2 < _ROWS_PER_W:
            nb = (rr + 2) % _NBUF
            in_cp[rr + 2] = pltpu.async_copy(
                in_hbm.at[base + rr + 2], bufs[nb], isems[nb])

        if rr < _ROWS_PER_W - 1:
            _scale(buf, inv, 0, _COLS)
            out_cp[rr] = pltpu.async_copy(buf, out_hbm.at[base + rr], osems[b])
        else:
            _scale(buf, inv, 0, _HALF)
            last_a = pltpu.async_copy(
                buf.at[pl.ds(0, _HALF)],
                out_hbm.at[base + rr, pl.ds(0, _HALF)], osems[b])
            _scale(buf, inv, _HALF, _COLS)
            last_b = pltpu.async_copy(
                buf.at[pl.ds(_HALF, _HALF)],
                out_hbm.at[base + rr, pl.ds(_HALF, _HALF)], isems[0])
            last_a.wait()
            last_b.wait()


def kernel(input, array):
    mesh = plsc.VectorSubcoreMesh(core_axis_name="c", subcore_axis_name="s")
    f = pl.kernel(
        _body,
        mesh=mesh,
        out_type=jax.ShapeDtypeStruct((_ROWS, _COLS), jnp.float32),
        scratch_types=[
            pltpu.VMEM((_COLS,), jnp.float32),
            pltpu.VMEM((_COLS,), jnp.float32),
            pltpu.VMEM((_COLS,), jnp.float32),
            pltpu.VMEM((256,), jnp.float32),
            pltpu.SemaphoreType.DMA,
            pltpu.SemaphoreType.DMA,
            pltpu.SemaphoreType.DMA,
            pltpu.SemaphoreType.DMA,
            pltpu.SemaphoreType.DMA,
            pltpu.SemaphoreType.DMA,
        ],
        compiler_params=pltpu.CompilerParams(needs_layout_passes=False),
    )
    return f(input, array)
